# initial kernel scaffold (unmeasured)
import functools

import jax
import jax.numpy as jnp
from jax import lax
from jax.experimental import pallas as pl
from jax.experimental.pallas import tpu as pltpu

N_DEV = 8
TAPS = 4
HALO = TAPS - 1


def kernel(x, k):
    b, s, c = x.shape
    assert k.shape == (TAPS, c)

    def body(x_ref, k_ref, out_ref, halo_ref, send_ref, send_sem, recv_sem):
        my_i = lax.axis_index("i")
        left = lax.rem(my_i - 1 + N_DEV, N_DEV)
        right = lax.rem(my_i + 1, N_DEV)

        barrier_sem = pltpu.get_barrier_semaphore()
        for nbr in (left, right):
            pl.semaphore_signal(
                barrier_sem, inc=1,
                device_id=(nbr,), device_id_type=pl.DeviceIdType.MESH,
            )
        pl.semaphore_wait(barrier_sem, 2)

        send_ref[...] = x_ref[:, s - HALO:, :]
        rdma = pltpu.make_async_remote_copy(
            src_ref=send_ref,
            dst_ref=halo_ref,
            send_sem=send_sem,
            recv_sem=recv_sem,
            device_id=(right,),
            device_id_type=pl.DeviceIdType.MESH,
        )

        @pl.when(my_i < N_DEV - 1)
        def _():
            rdma.start()

        @pl.when(my_i == 0)
        def _():
            halo_ref[...] = jnp.zeros((b, HALO, c), dtype=halo_ref.dtype)

        @pl.when(my_i > 0)
        def _():
            rdma.wait_recv()

        @pl.when(my_i < N_DEV - 1)
        def _():
            rdma.wait_send()

        for bi in range(b):
            xb = x_ref[bi]
            pad = jnp.concatenate([halo_ref[bi], xb], axis=0)
            acc = xb * k_ref[TAPS - 1]
            for t in range(TAPS - 1):
                acc = acc + pad[t:t + s, :] * k_ref[t]
            out_ref[bi] = (acc * jax.nn.sigmoid(acc)).astype(out_ref.dtype)

        @functools.partial(pl.run_scoped, exit_sem=pltpu.SemaphoreType.REGULAR)
        def _(exit_sem):
            for nbr in (left, right):
                pl.semaphore_signal(
                    exit_sem, inc=1,
                    device_id=(nbr,), device_id_type=pl.DeviceIdType.MESH,
                )
            pl.semaphore_wait(exit_sem, 2)

    return pl.pallas_call(
        body,
        out_shape=jax.ShapeDtypeStruct((b, s, c), jnp.bfloat16),
        in_specs=[
            pl.BlockSpec(memory_space=pltpu.VMEM),
            pl.BlockSpec(memory_space=pltpu.VMEM),
        ],
        out_specs=pl.BlockSpec(memory_space=pltpu.VMEM),
        scratch_shapes=[
            pltpu.VMEM((b, HALO, c), x.dtype),
            pltpu.VMEM((b, HALO, c), x.dtype),
            pltpu.SemaphoreType.DMA,
            pltpu.SemaphoreType.DMA,
        ],
        compiler_params=pltpu.CompilerParams(collective_id=0),
    )(x, k)


# baseline (device time: 61077 ns/iter reference)
import functools

import jax
import jax.numpy as jnp
from jax import lax
from jax.experimental import pallas as pl
from jax.experimental.pallas import tpu as pltpu

N_DEV = 8
TAPS = 4
HALO = TAPS - 1


def kernel(x, k):
    b, s, c = x.shape
    assert k.shape == (TAPS, c)

    def body(x_ref, k_ref, out_ref, halo_ref, send_ref, send_sem, recv_sem):
        my_i = lax.axis_index("i")
        left = lax.rem(my_i - 1 + N_DEV, N_DEV)
        right = lax.rem(my_i + 1, N_DEV)

        barrier_sem = pltpu.get_barrier_semaphore()
        for nbr in (left, right):
            pl.semaphore_signal(
                barrier_sem, inc=1,
                device_id=(nbr,), device_id_type=pl.DeviceIdType.MESH,
            )
        pl.semaphore_wait(barrier_sem, 2)

        send_ref[...] = x_ref[:, s - HALO:, :]
        rdma = pltpu.make_async_remote_copy(
            src_ref=send_ref,
            dst_ref=halo_ref,
            send_sem=send_sem,
            recv_sem=recv_sem,
            device_id=(right,),
            device_id_type=pl.DeviceIdType.MESH,
        )

        @pl.when(my_i < N_DEV - 1)
        def _():
            rdma.start()

        @pl.when(my_i == 0)
        def _():
            halo_ref[...] = jnp.zeros((b, HALO, c), dtype=halo_ref.dtype)

        @pl.when(my_i > 0)
        def _():
            rdma.wait_recv()

        @pl.when(my_i < N_DEV - 1)
        def _():
            rdma.wait_send()

        for bi in range(b):
            xb = x_ref[bi]
            pad = jnp.concatenate([halo_ref[bi], xb], axis=0)
            acc = xb * k_ref[TAPS - 1]
            for t in range(TAPS - 1):
                acc = acc + pad[t:t + s, :] * k_ref[t]
            out_ref[bi] = (acc * jax.nn.sigmoid(acc)).astype(out_ref.dtype)

        @functools.partial(pl.run_scoped, exit_sem=pltpu.SemaphoreType.REGULAR)
        def _(exit_sem):
            for nbr in (left, right):
                pl.semaphore_signal(
                    exit_sem, inc=1,
                    device_id=(nbr,), device_id_type=pl.DeviceIdType.MESH,
                )
            pl.semaphore_wait(exit_sem, 2)

    return pl.pallas_call(
        body,
        out_shape=jax.ShapeDtypeStruct((b, s, c), jnp.bfloat16),
        in_specs=[
            pl.BlockSpec(memory_space=pltpu.VMEM),
            pl.BlockSpec(memory_space=pltpu.VMEM),
        ],
        out_specs=pl.BlockSpec(memory_space=pltpu.VMEM),
        scratch_shapes=[
            pltpu.VMEM((b, HALO, c), x.dtype),
            pltpu.VMEM((b, HALO, c), x.dtype),
            pltpu.SemaphoreType.DMA,
            pltpu.SemaphoreType.DMA,
        ],
        compiler_params=pltpu.CompilerParams(
            collective_id=0,
            vmem_limit_bytes=110 * 1024 * 1024,
        ),
    )(x, k)


# device time: 50958 ns/iter; 1.1986x vs baseline; 1.1986x over previous
import functools

import jax
import jax.numpy as jnp
from jax import lax
from jax.experimental import pallas as pl
from jax.experimental.pallas import tpu as pltpu

N_DEV = 8
TAPS = 4
HALO = TAPS - 1


def kernel(x, k):
    b, s, c = x.shape
    assert k.shape == (TAPS, c)

    def body(x_ref, x_hbm_ref, k_ref, out_ref, halo_ref, send_sem, recv_sem):
        my_i = lax.axis_index("i")
        left = lax.rem(my_i - 1 + N_DEV, N_DEV)
        right = lax.rem(my_i + 1, N_DEV)
        pid = pl.program_id(0)

        rdma = pltpu.make_async_remote_copy(
            src_ref=x_hbm_ref.at[:, pl.ds(s - HALO, HALO), :],
            dst_ref=halo_ref,
            send_sem=send_sem,
            recv_sem=recv_sem,
            device_id=(right,),
            device_id_type=pl.DeviceIdType.MESH,
        )

        @pl.when(pid == 0)
        def _():
            barrier_sem = pltpu.get_barrier_semaphore()
            for nbr in (left, right):
                pl.semaphore_signal(
                    barrier_sem, inc=1,
                    device_id=(nbr,), device_id_type=pl.DeviceIdType.MESH,
                )
            pl.semaphore_wait(barrier_sem, 2)

            @pl.when(my_i < N_DEV - 1)
            def _():
                rdma.start()

            @pl.when(my_i == 0)
            def _():
                halo_ref[...] = jnp.zeros((b, HALO, c), dtype=halo_ref.dtype)

            @pl.when(my_i > 0)
            def _():
                rdma.wait_recv()

            @pl.when(my_i < N_DEV - 1)
            def _():
                rdma.wait_send()

        xb = x_ref[0]
        hb = halo_ref[pid]
        pad = jnp.concatenate([hb, xb], axis=0)
        acc = xb * k_ref[TAPS - 1]
        for t in range(TAPS - 1):
            acc = acc + pad[t:t + s, :] * k_ref[t]
        out_ref[0] = (acc * jax.nn.sigmoid(acc)).astype(out_ref.dtype)

        @pl.when(pid == b - 1)
        def _():
            @functools.partial(
                pl.run_scoped, exit_sem=pltpu.SemaphoreType.REGULAR
            )
            def _(exit_sem):
                for nbr in (left, right):
                    pl.semaphore_signal(
                        exit_sem, inc=1,
                        device_id=(nbr,), device_id_type=pl.DeviceIdType.MESH,
                    )
                pl.semaphore_wait(exit_sem, 2)

    return pl.pallas_call(
        body,
        grid=(b,),
        out_shape=jax.ShapeDtypeStruct((b, s, c), jnp.bfloat16),
        in_specs=[
            pl.BlockSpec((1, s, c), lambda i: (i, 0, 0)),
            pl.BlockSpec(memory_space=pl.ANY),
            pl.BlockSpec(memory_space=pltpu.VMEM),
        ],
        out_specs=pl.BlockSpec((1, s, c), lambda i: (i, 0, 0)),
        scratch_shapes=[
            pltpu.VMEM((b, HALO, c), x.dtype),
            pltpu.SemaphoreType.DMA,
            pltpu.SemaphoreType.DMA,
        ],
        compiler_params=pltpu.CompilerParams(
            collective_id=0,
            vmem_limit_bytes=110 * 1024 * 1024,
        ),
    )(x, x, k)


# device time: 44130 ns/iter; 1.3840x vs baseline; 1.1547x over previous
import functools

import jax
import jax.numpy as jnp
from jax import lax
from jax.experimental import pallas as pl
from jax.experimental.pallas import tpu as pltpu

N_DEV = 8
TAPS = 4
HALO = TAPS - 1


def kernel(x, k):
    b, s, c = x.shape
    assert k.shape == (TAPS, c)

    def body(x_ref, x_hbm_ref, k_ref, out_ref, halo_ref, send_sem, recv_sem):
        my_i = lax.axis_index("i")
        left = lax.rem(my_i - 1 + N_DEV, N_DEV)
        right = lax.rem(my_i + 1, N_DEV)
        pid = pl.program_id(0)

        rdma = pltpu.make_async_remote_copy(
            src_ref=x_hbm_ref.at[:, pl.ds(s - HALO, HALO), :],
            dst_ref=halo_ref,
            send_sem=send_sem,
            recv_sem=recv_sem,
            device_id=(right,),
            device_id_type=pl.DeviceIdType.MESH,
        )

        @pl.when(pid == 0)
        def _():
            barrier_sem = pltpu.get_barrier_semaphore()
            for nbr in (left, right):
                pl.semaphore_signal(
                    barrier_sem, inc=1,
                    device_id=(nbr,), device_id_type=pl.DeviceIdType.MESH,
                )
            pl.semaphore_wait(barrier_sem, 2)

            @pl.when(my_i < N_DEV - 1)
            def _():
                rdma.start()

            @pl.when(my_i == 0)
            def _():
                halo_ref[...] = jnp.zeros((b, HALO, c), dtype=halo_ref.dtype)

            @pl.when(my_i > 0)
            def _():
                rdma.wait_recv()

            @pl.when(my_i < N_DEV - 1)
            def _():
                rdma.wait_send()

        xb = x_ref[0].astype(jnp.bfloat16)
        hb = halo_ref[pid].astype(jnp.bfloat16)
        kb = k_ref[...].astype(jnp.bfloat16)
        pad = jnp.concatenate([hb, xb], axis=0)
        acc = xb * kb[TAPS - 1]
        for t in range(TAPS - 1):
            acc = acc + pad[t:t + s, :] * kb[t]
        out_ref[0] = (acc * jax.nn.sigmoid(acc)).astype(out_ref.dtype)

        @pl.when(pid == b - 1)
        def _():
            @functools.partial(
                pl.run_scoped, exit_sem=pltpu.SemaphoreType.REGULAR
            )
            def _(exit_sem):
                for nbr in (left, right):
                    pl.semaphore_signal(
                        exit_sem, inc=1,
                        device_id=(nbr,), device_id_type=pl.DeviceIdType.MESH,
                    )
                pl.semaphore_wait(exit_sem, 2)

    return pl.pallas_call(
        body,
        grid=(b,),
        out_shape=jax.ShapeDtypeStruct((b, s, c), jnp.bfloat16),
        in_specs=[
            pl.BlockSpec((1, s, c), lambda i: (i, 0, 0)),
            pl.BlockSpec(memory_space=pl.ANY),
            pl.BlockSpec(memory_space=pltpu.VMEM),
        ],
        out_specs=pl.BlockSpec((1, s, c), lambda i: (i, 0, 0)),
        scratch_shapes=[
            pltpu.VMEM((b, HALO, c), x.dtype),
            pltpu.SemaphoreType.DMA,
            pltpu.SemaphoreType.DMA,
        ],
        compiler_params=pltpu.CompilerParams(
            collective_id=0,
            vmem_limit_bytes=110 * 1024 * 1024,
        ),
    )(x, x, k)


# device time: 31494 ns/iter; 1.9393x vs baseline; 1.4012x over previous
import jax
import jax.numpy as jnp
from jax import lax
from jax.experimental import pallas as pl
from jax.experimental.pallas import tpu as pltpu

N_DEV = 8
TAPS = 4
HALO = TAPS - 1


def _halo_exchange(x_tail):
    b, h, c = x_tail.shape

    def body(tail_ref, halo_ref, send_sem, recv_sem):
        my_i = lax.axis_index("i")
        left = lax.rem(my_i - 1 + N_DEV, N_DEV)
        right = lax.rem(my_i + 1, N_DEV)

        barrier_sem = pltpu.get_barrier_semaphore()
        for nbr in (left, right):
            pl.semaphore_signal(
                barrier_sem, inc=1,
                device_id=(nbr,), device_id_type=pl.DeviceIdType.MESH,
            )
        pl.semaphore_wait(barrier_sem, 2)

        rdma = pltpu.make_async_remote_copy(
            src_ref=tail_ref,
            dst_ref=halo_ref,
            send_sem=send_sem,
            recv_sem=recv_sem,
            device_id=(right,),
            device_id_type=pl.DeviceIdType.MESH,
        )

        @pl.when(my_i < N_DEV - 1)
        def _():
            rdma.start()

        @pl.when(my_i == 0)
        def _():
            halo_ref[...] = jnp.zeros((b, h, c), dtype=halo_ref.dtype)

        @pl.when(my_i > 0)
        def _():
            rdma.wait_recv()

        @pl.when(my_i < N_DEV - 1)
        def _():
            rdma.wait_send()

        import functools

        @functools.partial(pl.run_scoped, exit_sem=pltpu.SemaphoreType.REGULAR)
        def _(exit_sem):
            for nbr in (left, right):
                pl.semaphore_signal(
                    exit_sem, inc=1,
                    device_id=(nbr,), device_id_type=pl.DeviceIdType.MESH,
                )
            pl.semaphore_wait(exit_sem, 2)

    return pl.pallas_call(
        body,
        out_shape=jax.ShapeDtypeStruct((b, h, c), x_tail.dtype),
        in_specs=[pl.BlockSpec(memory_space=pltpu.MemorySpace.VMEM)],
        out_specs=pl.BlockSpec(memory_space=pltpu.MemorySpace.VMEM),
        scratch_shapes=[
            pltpu.SemaphoreType.DMA,
            pltpu.SemaphoreType.DMA,
        ],
        compiler_params=pltpu.CompilerParams(collective_id=0),
    )(x_tail)


def _conv_silu(x, k, halo):
    b, s, c = x.shape

    def body(x_ref, k_ref, halo_ref, out_ref):
        pid = pl.program_id(0)
        xb = x_ref[0].astype(jnp.bfloat16)
        kb = k_ref[...].astype(jnp.bfloat16)
        hb = halo_ref[pid].astype(jnp.bfloat16)
        pad = jnp.concatenate([hb, xb], axis=0)
        acc = xb * kb[TAPS - 1]
        for t in range(TAPS - 1):
            acc = acc + pad[t:t + s, :] * kb[t]
        out_ref[0] = (acc * jax.nn.sigmoid(acc)).astype(out_ref.dtype)

    return pl.pallas_call(
        body,
        grid=(b,),
        out_shape=jax.ShapeDtypeStruct((b, s, c), jnp.bfloat16),
        in_specs=[
            pl.BlockSpec((1, s, c), lambda i: (i, 0, 0)),
            pl.BlockSpec(memory_space=pltpu.MemorySpace.VMEM),
            pl.BlockSpec(memory_space=pltpu.MemorySpace.VMEM),
        ],
        out_specs=pl.BlockSpec((1, s, c), lambda i: (i, 0, 0)),
        compiler_params=pltpu.CompilerParams(
            vmem_limit_bytes=60 * 1024 * 1024,
        ),
    )(x, k, halo)


def kernel(x, k):
    b, s, c = x.shape
    assert k.shape == (TAPS, c)
    x_tail = lax.slice(x, (0, s - HALO, 0), (b, s, c))
    halo = _halo_exchange(x_tail)
    return _conv_silu(x, k, halo)
